# baseline (speedup 1.0000x reference)
import jax
import jax.numpy as jnp
from jax import lax
from jax.experimental import pallas as pl
from jax.experimental.pallas import tpu as pltpu
from jax.experimental.pallas import tpu_sc as plsc

EPS = 1e-5
G = 128
NW = 32
BE = 2048


def _node_gemm(P, Wd_t, Ws_t, n_real):
    k, np_, din = P.shape
    mid = Wd_t.shape[1]

    def body(p_ref, wd_ref, ws_ref, a_ref, b_ref):
        H = jnp.sum(p_ref[...], axis=0)
        rows = lax.broadcasted_iota(jnp.int32, (np_, 1), 0)
        H = jnp.where(rows < n_real, H, 0.0)
        a_ref[...] = jnp.dot(H, wd_ref[...], preferred_element_type=jnp.float32)
        b_ref[...] = jnp.dot(H, ws_ref[...], preferred_element_type=jnp.float32)

    return pl.pallas_call(
        body,
        out_shape=[jax.ShapeDtypeStruct((np_, mid), jnp.float32)] * 2,
    )(P, Wd_t, Ws_t)


def _sc_gather(A, B, dsti, srci, e_pad):
    np_, mid = A.shape
    e_pt = e_pad // NW
    ng = e_pt // G
    nj = mid // 16
    mesh = plsc.VectorSubcoreMesh(core_axis_name="c", subcore_axis_name="s")

    def body(a_hbm, b_hbm, di_hbm, si_hbm, h1_hbm, s1_hbm, q1_hbm,
             dstb, srcb, ab0, ab1, bb0, bb1, hb0, hb1, stg,
             sga0, sga1, sgb0, sgb1, swb0, swb1):
        c = lax.axis_index("c")
        s = lax.axis_index("s")
        wid = s * 2 + c
        pltpu.sync_copy(di_hbm.at[wid], dstb)
        pltpu.sync_copy(si_hbm.at[wid], srcb)
        zero = jnp.zeros((16,), jnp.float32)

        def group(g, carry):
            cp1 = pltpu.async_copy(a_hbm.at[dstb.at[g]], ab0, sga0)
            cp2 = pltpu.async_copy(b_hbm.at[srcb.at[g]], bb0, sgb0)
            cp1.wait()
            cp2.wait()

            def row(r, rc):
                rss, rqq = rc
                nss = []
                nqq = []
                for j in range(nj):
                    sl = pl.ds(j * 16, 16)
                    h = ab0[r, sl] + bb0[r, sl]
                    ab0[r, sl] = h
                    nss.append(rss[j] + h)
                    nqq.append(rqq[j] + h * h)
                return (tuple(nss), tuple(nqq))

            carry = lax.fori_loop(0, G, row, carry, unroll=8)
            pltpu.sync_copy(ab0, h1_hbm.at[pl.ds(wid * e_pt + g * G, G)])
            return carry

        init = (tuple(zero for _ in range(nj)), tuple(zero for _ in range(nj)))
        ss, qq = lax.fori_loop(0, ng, group, init)
        for j in range(nj):
            stg[pl.ds(j * 16, 16)] = ss[j]
        pltpu.sync_copy(stg, s1_hbm.at[wid])
        for j in range(nj):
            stg[pl.ds(j * 16, 16)] = qq[j]
        pltpu.sync_copy(stg, q1_hbm.at[wid])

    f = pl.kernel(
        body,
        out_type=[jax.ShapeDtypeStruct((e_pad, mid), jnp.float32),
                  jax.ShapeDtypeStruct((NW, mid), jnp.float32),
                  jax.ShapeDtypeStruct((NW, mid), jnp.float32)],
        mesh=mesh,
        compiler_params=pltpu.CompilerParams(use_tc_tiling_on_sc=False),
        scratch_types=[
            pltpu.VMEM((ng, G), jnp.int32),
            pltpu.VMEM((ng, G), jnp.int32),
            pltpu.VMEM((G, mid), jnp.float32),
            pltpu.VMEM((G, mid), jnp.float32),
            pltpu.VMEM((G, mid), jnp.float32),
            pltpu.VMEM((G, mid), jnp.float32),
            pltpu.VMEM((G, mid), jnp.float32),
            pltpu.VMEM((G, mid), jnp.float32),
            pltpu.VMEM((mid,), jnp.float32),
            pltpu.SemaphoreType.DMA,
            pltpu.SemaphoreType.DMA,
            pltpu.SemaphoreType.DMA,
            pltpu.SemaphoreType.DMA,
            pltpu.SemaphoreType.DMA,
            pltpu.SemaphoreType.DMA,
        ],
    )
    return f(A, B, dsti, srci)


def _edge_mlp(h1, sc1, sh1, W2t, e_real):
    e_pad, mid = h1.shape
    dout = W2t.shape[1]
    nblk = e_pad // BE

    def body(h1_ref, sc_ref, sh_ref, w_ref, h2_ref, st_ref):
        i = pl.program_id(0)
        r = jnp.maximum(h1_ref[...] * sc_ref[...] + sh_ref[...], 0.0)
        h2 = jnp.dot(r, w_ref[...], preferred_element_type=jnp.float32)
        h2_ref[...] = h2
        rows = lax.broadcasted_iota(jnp.int32, (BE, 1), 0) + i * BE
        hm = jnp.where(rows < e_real, h2, 0.0)
        s = jnp.sum(hm, axis=0, keepdims=True)
        q = jnp.sum(hm * hm, axis=0, keepdims=True)
        sq = jnp.concatenate([s, q, jnp.zeros((6, dout), jnp.float32)], axis=0)

        @pl.when(i == 0)
        def _():
            st_ref[...] = sq

        @pl.when(i > 0)
        def _():
            st_ref[...] = st_ref[...] + sq

    return pl.pallas_call(
        body,
        grid=(nblk,),
        in_specs=[pl.BlockSpec((BE, mid), lambda i: (i, 0)),
                  pl.BlockSpec((1, mid), lambda i: (0, 0)),
                  pl.BlockSpec((1, mid), lambda i: (0, 0)),
                  pl.BlockSpec((mid, dout), lambda i: (0, 0))],
        out_specs=[pl.BlockSpec((BE, dout), lambda i: (i, 0)),
                   pl.BlockSpec((8, dout), lambda i: (0, 0))],
        out_shape=[jax.ShapeDtypeStruct((e_pad, dout), jnp.float32),
                   jax.ShapeDtypeStruct((8, dout), jnp.float32)],
    )(h1, sc1.reshape(1, mid), sh1.reshape(1, mid), W2t)


def _sc_scatter(h2, dsti, sc2, sh2, zr, n_pad):
    e_pad, dout = h2.shape
    e_pt = e_pad // NW
    ng = e_pt // G
    nj = dout // 16
    zrows = n_pad // 16
    mesh = plsc.VectorSubcoreMesh(core_axis_name="c", subcore_axis_name="s")

    def body(h2_hbm, di_hbm, sc_hbm, sh_hbm, zr_hbm, p_hbm,
             dstb, hb0, hb1, scb, shb, acc, sld0, sld1):
        c = lax.axis_index("c")
        s = lax.axis_index("s")
        wid = s * 2 + c
        pltpu.sync_copy(zr_hbm, acc.at[pl.ds(s * zrows, zrows)])
        pltpu.sync_copy(di_hbm.at[wid], dstb)
        pltpu.sync_copy(sc_hbm, scb)
        pltpu.sync_copy(sh_hbm, shb)
        scv = tuple(scb[pl.ds(j * 16, 16)] for j in range(nj))
        shv = tuple(shb[pl.ds(j * 16, 16)] for j in range(nj))
        plsc.subcore_barrier()
        def group(g, _):
            pltpu.sync_copy(h2_hbm.at[pl.ds((wid * ng + g) * G, G)], hb0)

            def row(r, _2):
                for j in range(nj):
                    sl = pl.ds(j * 16, 16)
                    hb0[r, sl] = jnp.maximum(
                        hb0[r, sl] * scv[j] + shv[j], 0.0)
                return 0

            lax.fori_loop(0, G, row, 0, unroll=8)
            pltpu.sync_copy(hb0, acc.at[dstb.at[g]], add=True)
            return 0

        lax.fori_loop(0, ng, group, 0)
        plsc.subcore_barrier()
        pltpu.sync_copy(acc.at[pl.ds(s * zrows, zrows)],
                        p_hbm.at[c, pl.ds(s * zrows, zrows)])

    f = pl.kernel(
        body,
        out_type=jax.ShapeDtypeStruct((2, n_pad, dout), jnp.float32),
        mesh=mesh,
        compiler_params=pltpu.CompilerParams(use_tc_tiling_on_sc=False),
        scratch_types=[
            pltpu.VMEM((ng, G), jnp.int32),
            pltpu.VMEM((G, dout), jnp.float32),
            pltpu.VMEM((G, dout), jnp.float32),
            pltpu.VMEM((dout,), jnp.float32),
            pltpu.VMEM((dout,), jnp.float32),
            pltpu.VMEM_SHARED((n_pad, dout), jnp.float32),
            pltpu.SemaphoreType.DMA,
            pltpu.SemaphoreType.DMA,
        ],
    )
    return f(h2, dsti, sc2, sh2, zr)


def _final_a(Ps, W1t, n_real):
    np_ = Ps[0].shape[1]
    BR = 1280
    nblk = np_ // BR
    dz = sum(int(p.shape[2]) for p in Ps)
    dh = W1t.shape[1]

    def body(*refs):
        p_refs = refs[:6]
        w_ref, y_ref, st_ref = refs[6], refs[7], refs[8]
        i = pl.program_id(0)
        z = jnp.concatenate([r[...][0] + r[...][1] for r in p_refs], axis=1)
        y = jnp.dot(z, w_ref[...], preferred_element_type=jnp.float32)
        y_ref[...] = y
        rows = lax.broadcasted_iota(jnp.int32, (BR, 1), 0) + i * BR
        ym = jnp.where(rows < n_real, y, 0.0)
        s = jnp.sum(ym, axis=0, keepdims=True)
        q = jnp.sum(ym * ym, axis=0, keepdims=True)
        sq = jnp.concatenate([s, q, jnp.zeros((6, dh), jnp.float32)], axis=0)

        @pl.when(i == 0)
        def _():
            st_ref[...] = sq

        @pl.when(i > 0)
        def _():
            st_ref[...] = st_ref[...] + sq

    in_specs = [pl.BlockSpec((2, BR, int(p.shape[2])), lambda i: (0, i, 0))
                for p in Ps]
    in_specs.append(pl.BlockSpec((dz, dh), lambda i: (0, 0)))
    return pl.pallas_call(
        body,
        grid=(nblk,),
        in_specs=in_specs,
        out_specs=[pl.BlockSpec((BR, dh), lambda i: (i, 0)),
                   pl.BlockSpec((8, dh), lambda i: (0, 0))],
        out_shape=[jax.ShapeDtypeStruct((np_, dh), jnp.float32),
                   jax.ShapeDtypeStruct((8, dh), jnp.float32)],
    )(*Ps, W1t)


def _final_b(y1, sc, sh, W2t, b2, W3t, b3):
    np_, dh = y1.shape
    d2 = W2t.shape[1]
    BR = 1280
    nblk = np_ // BR

    def body(y_ref, sc_ref, sh_ref, w2_ref, b2_ref, w3_ref, b3_ref, o_ref):
        y = jnp.maximum(y_ref[...] * sc_ref[...] + sh_ref[...], 0.0)
        h = jnp.maximum(
            jnp.dot(y, w2_ref[...], preferred_element_type=jnp.float32)
            + b2_ref[...], 0.0)
        o = jnp.dot(h, w3_ref[...], preferred_element_type=jnp.float32) + b3_ref[...]
        o_ref[...] = jax.nn.sigmoid(o)

    return pl.pallas_call(
        body,
        grid=(nblk,),
        in_specs=[pl.BlockSpec((BR, dh), lambda i: (i, 0)),
                  pl.BlockSpec((1, dh), lambda i: (0, 0)),
                  pl.BlockSpec((1, dh), lambda i: (0, 0)),
                  pl.BlockSpec((dh, d2), lambda i: (0, 0)),
                  pl.BlockSpec((1, d2), lambda i: (0, 0)),
                  pl.BlockSpec((d2, 1), lambda i: (0, 0)),
                  pl.BlockSpec((1, 1), lambda i: (0, 0))],
        out_specs=pl.BlockSpec((BR, 1), lambda i: (i, 0)),
        out_shape=jax.ShapeDtypeStruct((np_, 1), jnp.float32),
    )(y1, sc.reshape(1, dh), sh.reshape(1, dh), W2t, b2.reshape(1, d2),
      W3t, b3.reshape(1, 1))


def _bn_affine(s, q, cnt, g, be):
    mu = s / cnt
    var = q / cnt - mu * mu
    inv = g * lax.rsqrt(var + EPS)
    return inv, be - mu * inv


def kernel(x, edge_index, params):
    N, D = x.shape
    E = edge_index.shape[1]
    NP = ((N + 2) + 1279) // 1280 * 1280
    e_pad = (E + NW * G - 1) // (NW * G) * (NW * G)
    e_pt = e_pad // NW
    ng = e_pt // G
    zrows = NP // 16

    src = edge_index[0]
    dst = edge_index[1]
    pad_e = e_pad - E
    fill = jnp.full((pad_e,), N, jnp.int32)
    dsti = jnp.concatenate([dst, fill]).reshape(NW, ng, G)
    srci = jnp.concatenate([src, fill]).reshape(NW, ng, G)
    x_pad = jnp.pad(x, ((0, NP - N), (0, 0)))

    zr_cache = {}
    Ps = []
    P_stack = x_pad[None]
    for p in params['convs']:
        d = P_stack.shape[2]
        Wd_t = (p['W1'][:, :d] - p['W1'][:, d:]).T
        Ws_t = p['W1'][:, d:].T
        A, Bm = _node_gemm(P_stack, Wd_t, Ws_t, N)
        h1, s1p, q1p = _sc_gather(A, Bm, dsti, srci, e_pad)
        sc1, sh1 = _bn_affine(s1p.sum(0), q1p.sum(0), E, p['g1'], p['be1'])
        h2, st = _edge_mlp(h1, sc1, sh1, p['W2'].T, E)
        sc2, sh2 = _bn_affine(st[0], st[1], E, p['g2'], p['be2'])
        dout = p['W2'].shape[0]
        if dout not in zr_cache:
            zr_cache[dout] = jnp.zeros((zrows, dout), jnp.float32)
        P_stack = _sc_scatter(h2, dsti, sc2, sh2, zr_cache[dout], NP)
        Ps.append(P_stack)

    y1, stf = _final_a(Ps, params['seq1']['W'].T, N)
    scf, shf = _bn_affine(stf[0], stf[1], N, params['seq1']['g'],
                          params['seq1']['be'])
    out = _final_b(y1, scf, shf, params['seq2']['W'].T, params['seq2']['b'],
                   params['lin']['W'].T, params['lin']['b'])
    return out[:N]


# speedup vs baseline: 1.1199x; 1.1199x over previous
import jax
import jax.numpy as jnp
from jax import lax
from jax.experimental import pallas as pl
from jax.experimental.pallas import tpu as pltpu
from jax.experimental.pallas import tpu_sc as plsc

EPS = 1e-5
G = 128
NW = 32
BE = 2048


def _node_gemm(P, Wd_t, Ws_t, n_real):
    k, np_, din = P.shape
    mid = Wd_t.shape[1]

    def body(p_ref, wd_ref, ws_ref, a_ref, b_ref):
        H = jnp.sum(p_ref[...], axis=0)
        rows = lax.broadcasted_iota(jnp.int32, (np_, 1), 0)
        H = jnp.where(rows < n_real, H, 0.0)
        a_ref[...] = jnp.dot(H, wd_ref[...], preferred_element_type=jnp.float32)
        b_ref[...] = jnp.dot(H, ws_ref[...], preferred_element_type=jnp.float32)

    return pl.pallas_call(
        body,
        out_shape=[jax.ShapeDtypeStruct((np_, mid), jnp.float32)] * 2,
    )(P, Wd_t, Ws_t)


def _sc_gather(A, B, dsti, srci, e_pad):
    np_, mid = A.shape
    e_pt = e_pad // NW
    ng = e_pt // G
    nj = mid // 16
    mesh = plsc.VectorSubcoreMesh(core_axis_name="c", subcore_axis_name="s")

    def body(a_hbm, b_hbm, di_hbm, si_hbm, h1_hbm, s1_hbm, q1_hbm,
             dstb, srcb, ab0, bb0, stg, sga0, sgb0):
        c = lax.axis_index("c")
        s = lax.axis_index("s")
        wid = s * 2 + c
        pltpu.sync_copy(di_hbm.at[wid], dstb)
        pltpu.sync_copy(si_hbm.at[wid], srcb)
        zero = jnp.zeros((16,), jnp.float32)

        init = (tuple(zero for _ in range(nj)), tuple(zero for _ in range(nj)))

        def group(g, carry):
            cp1 = pltpu.async_copy(a_hbm.at[dstb.at[g]], ab0, sga0)
            cp2 = pltpu.async_copy(b_hbm.at[srcb.at[g]], bb0, sgb0)
            cp1.wait()
            cp2.wait()

            def row(r, rc):
                rss, rqq = rc
                nss = []
                nqq = []
                for j in range(nj):
                    sl = pl.ds(j * 16, 16)
                    h = ab0[r, sl] + bb0[r, sl]
                    ab0[r, sl] = h
                    nss.append(rss[j] + h)
                    nqq.append(rqq[j] + h * h)
                return (tuple(nss), tuple(nqq))

            carry = lax.fori_loop(0, G, row, carry)
            pltpu.sync_copy(ab0, h1_hbm.at[pl.ds(wid * e_pt + g * G, G)])
            return carry

        ss, qq = lax.fori_loop(0, ng, group, init)
        for j in range(nj):
            stg[pl.ds(j * 16, 16)] = ss[j]
        pltpu.sync_copy(stg, s1_hbm.at[wid])
        for j in range(nj):
            stg[pl.ds(j * 16, 16)] = qq[j]
        pltpu.sync_copy(stg, q1_hbm.at[wid])

    f = pl.kernel(
        body,
        out_type=[jax.ShapeDtypeStruct((e_pad, mid), jnp.float32),
                  jax.ShapeDtypeStruct((NW, mid), jnp.float32),
                  jax.ShapeDtypeStruct((NW, mid), jnp.float32)],
        mesh=mesh,
        compiler_params=pltpu.CompilerParams(use_tc_tiling_on_sc=False),
        scratch_types=[
            pltpu.VMEM((ng, G), jnp.int32),
            pltpu.VMEM((ng, G), jnp.int32),
            pltpu.VMEM((G, mid), jnp.float32),
            pltpu.VMEM((G, mid), jnp.float32),
            pltpu.VMEM((mid,), jnp.float32),
            pltpu.SemaphoreType.DMA,
            pltpu.SemaphoreType.DMA,
        ],
    )
    return f(A, B, dsti, srci)


def _edge_mlp(h1, sc1, sh1, W2t, e_real):
    e_pad, mid = h1.shape
    dout = W2t.shape[1]
    nblk = e_pad // BE

    def body(h1_ref, sc_ref, sh_ref, w_ref, h2_ref, st_ref):
        i = pl.program_id(0)
        r = jnp.maximum(h1_ref[...] * sc_ref[...] + sh_ref[...], 0.0)
        h2 = jnp.dot(r, w_ref[...], preferred_element_type=jnp.float32)
        h2_ref[...] = h2
        rows = lax.broadcasted_iota(jnp.int32, (BE, 1), 0) + i * BE
        hm = jnp.where(rows < e_real, h2, 0.0)
        s = jnp.sum(hm, axis=0, keepdims=True)
        q = jnp.sum(hm * hm, axis=0, keepdims=True)
        sq = jnp.concatenate([s, q, jnp.zeros((6, dout), jnp.float32)], axis=0)

        @pl.when(i == 0)
        def _():
            st_ref[...] = sq

        @pl.when(i > 0)
        def _():
            st_ref[...] = st_ref[...] + sq

    return pl.pallas_call(
        body,
        grid=(nblk,),
        in_specs=[pl.BlockSpec((BE, mid), lambda i: (i, 0)),
                  pl.BlockSpec((1, mid), lambda i: (0, 0)),
                  pl.BlockSpec((1, mid), lambda i: (0, 0)),
                  pl.BlockSpec((mid, dout), lambda i: (0, 0))],
        out_specs=[pl.BlockSpec((BE, dout), lambda i: (i, 0)),
                   pl.BlockSpec((8, dout), lambda i: (0, 0))],
        out_shape=[jax.ShapeDtypeStruct((e_pad, dout), jnp.float32),
                   jax.ShapeDtypeStruct((8, dout), jnp.float32)],
    )(h1, sc1.reshape(1, mid), sh1.reshape(1, mid), W2t)


def _sc_scatter(h2, dsti, sc2, sh2, zr, n_pad):
    e_pad, dout = h2.shape
    e_pt = e_pad // NW
    ng = e_pt // G
    nj = dout // 16
    zrows = n_pad // 16
    mesh = plsc.VectorSubcoreMesh(core_axis_name="c", subcore_axis_name="s")

    def body(h2_hbm, di_hbm, sc_hbm, sh_hbm, zr_hbm, p_hbm,
             dstb, hb0, scb, shb, acc):
        c = lax.axis_index("c")
        s = lax.axis_index("s")
        wid = s * 2 + c
        pltpu.sync_copy(zr_hbm, acc.at[pl.ds(s * zrows, zrows)])
        pltpu.sync_copy(di_hbm.at[wid], dstb)
        pltpu.sync_copy(sc_hbm, scb)
        pltpu.sync_copy(sh_hbm, shb)
        scv = tuple(scb[pl.ds(j * 16, 16)] for j in range(nj))
        shv = tuple(shb[pl.ds(j * 16, 16)] for j in range(nj))
        plsc.subcore_barrier()

        def group(g, _):
            pltpu.sync_copy(h2_hbm.at[pl.ds((wid * ng + g) * G, G)], hb0)

            def row(r, _2):
                for j in range(nj):
                    sl = pl.ds(j * 16, 16)
                    hb0[r, sl] = jnp.maximum(
                        hb0[r, sl] * scv[j] + shv[j], 0.0)
                return 0

            lax.fori_loop(0, G, row, 0)
            pltpu.sync_copy(hb0, acc.at[dstb.at[g]], add=True)
            return 0

        lax.fori_loop(0, ng, group, 0)
        plsc.subcore_barrier()
        pltpu.sync_copy(acc.at[pl.ds(s * zrows, zrows)],
                        p_hbm.at[c, pl.ds(s * zrows, zrows)])

    f = pl.kernel(
        body,
        out_type=jax.ShapeDtypeStruct((2, n_pad, dout), jnp.float32),
        mesh=mesh,
        compiler_params=pltpu.CompilerParams(use_tc_tiling_on_sc=False),
        scratch_types=[
            pltpu.VMEM((ng, G), jnp.int32),
            pltpu.VMEM((G, dout), jnp.float32),
            pltpu.VMEM((dout,), jnp.float32),
            pltpu.VMEM((dout,), jnp.float32),
            pltpu.VMEM_SHARED((n_pad, dout), jnp.float32),
        ],
    )
    return f(h2, dsti, sc2, sh2, zr)


def _final_a(Ps, W1t, n_real):
    np_ = Ps[0].shape[1]
    BR = 1280
    nblk = np_ // BR
    dz = sum(int(p.shape[2]) for p in Ps)
    dh = W1t.shape[1]

    def body(*refs):
        p_refs = refs[:6]
        w_ref, y_ref, st_ref = refs[6], refs[7], refs[8]
        i = pl.program_id(0)
        z = jnp.concatenate([r[...][0] + r[...][1] for r in p_refs], axis=1)
        y = jnp.dot(z, w_ref[...], preferred_element_type=jnp.float32)
        y_ref[...] = y
        rows = lax.broadcasted_iota(jnp.int32, (BR, 1), 0) + i * BR
        ym = jnp.where(rows < n_real, y, 0.0)
        s = jnp.sum(ym, axis=0, keepdims=True)
        q = jnp.sum(ym * ym, axis=0, keepdims=True)
        sq = jnp.concatenate([s, q, jnp.zeros((6, dh), jnp.float32)], axis=0)

        @pl.when(i == 0)
        def _():
            st_ref[...] = sq

        @pl.when(i > 0)
        def _():
            st_ref[...] = st_ref[...] + sq

    in_specs = [pl.BlockSpec((2, BR, int(p.shape[2])), lambda i: (0, i, 0))
                for p in Ps]
    in_specs.append(pl.BlockSpec((dz, dh), lambda i: (0, 0)))
    return pl.pallas_call(
        body,
        grid=(nblk,),
        in_specs=in_specs,
        out_specs=[pl.BlockSpec((BR, dh), lambda i: (i, 0)),
                   pl.BlockSpec((8, dh), lambda i: (0, 0))],
        out_shape=[jax.ShapeDtypeStruct((np_, dh), jnp.float32),
                   jax.ShapeDtypeStruct((8, dh), jnp.float32)],
    )(*Ps, W1t)


def _final_b(y1, sc, sh, W2t, b2, W3t, b3):
    np_, dh = y1.shape
    d2 = W2t.shape[1]
    BR = 1280
    nblk = np_ // BR

    def body(y_ref, sc_ref, sh_ref, w2_ref, b2_ref, w3_ref, b3_ref, o_ref):
        y = jnp.maximum(y_ref[...] * sc_ref[...] + sh_ref[...], 0.0)
        h = jnp.maximum(
            jnp.dot(y, w2_ref[...], preferred_element_type=jnp.float32)
            + b2_ref[...], 0.0)
        o = jnp.dot(h, w3_ref[...], preferred_element_type=jnp.float32) + b3_ref[...]
        o_ref[...] = jax.nn.sigmoid(o)

    return pl.pallas_call(
        body,
        grid=(nblk,),
        in_specs=[pl.BlockSpec((BR, dh), lambda i: (i, 0)),
                  pl.BlockSpec((1, dh), lambda i: (0, 0)),
                  pl.BlockSpec((1, dh), lambda i: (0, 0)),
                  pl.BlockSpec((dh, d2), lambda i: (0, 0)),
                  pl.BlockSpec((1, d2), lambda i: (0, 0)),
                  pl.BlockSpec((d2, 1), lambda i: (0, 0)),
                  pl.BlockSpec((1, 1), lambda i: (0, 0))],
        out_specs=pl.BlockSpec((BR, 1), lambda i: (i, 0)),
        out_shape=jax.ShapeDtypeStruct((np_, 1), jnp.float32),
    )(y1, sc.reshape(1, dh), sh.reshape(1, dh), W2t, b2.reshape(1, d2),
      W3t, b3.reshape(1, 1))


def _bn_affine(s, q, cnt, g, be):
    mu = s / cnt
    var = q / cnt - mu * mu
    inv = g * lax.rsqrt(var + EPS)
    return inv, be - mu * inv


def kernel(x, edge_index, params):
    N, D = x.shape
    E = edge_index.shape[1]
    NP = ((N + 2) + 1279) // 1280 * 1280
    e_pad = (E + NW * G - 1) // (NW * G) * (NW * G)
    e_pt = e_pad // NW
    ng = e_pt // G
    zrows = NP // 16

    src = edge_index[0]
    dst = edge_index[1]
    pad_e = e_pad - E
    fill = jnp.full((pad_e,), N, jnp.int32)
    dsti = jnp.concatenate([dst, fill]).reshape(NW, ng, G)
    srci = jnp.concatenate([src, fill]).reshape(NW, ng, G)
    x_pad = jnp.pad(x, ((0, NP - N), (0, 0)))

    zr_cache = {}
    Ps = []
    P_stack = x_pad[None]
    for p in params['convs']:
        d = P_stack.shape[2]
        Wd_t = (p['W1'][:, :d] - p['W1'][:, d:]).T
        Ws_t = p['W1'][:, d:].T
        A, Bm = _node_gemm(P_stack, Wd_t, Ws_t, N)
        h1, s1p, q1p = _sc_gather(A, Bm, dsti, srci, e_pad)
        sc1, sh1 = _bn_affine(s1p.sum(0), q1p.sum(0), E, p['g1'], p['be1'])
        h2, st = _edge_mlp(h1, sc1, sh1, p['W2'].T, E)
        sc2, sh2 = _bn_affine(st[0], st[1], E, p['g2'], p['be2'])
        dout = p['W2'].shape[0]
        if dout not in zr_cache:
            zr_cache[dout] = jnp.zeros((zrows, dout), jnp.float32)
        P_stack = _sc_scatter(h2, dsti, sc2, sh2, zr_cache[dout], NP)
        Ps.append(P_stack)

    y1, stf = _final_a(Ps, params['seq1']['W'].T, N)
    scf, shf = _bn_affine(stf[0], stf[1], N, params['seq1']['g'],
                          params['seq1']['be'])
    out = _final_b(y1, scf, shf, params['seq2']['W'].T, params['seq2']['b'],
                   params['lin']['W'].T, params['lin']['b'])
    return out[:N]
